# baseline (speedup 1.0000x reference)
import jax
import jax.numpy as jnp
from jax import lax
from jax.experimental import pallas as pl
from jax.experimental.pallas import tpu as pltpu
from jax.experimental.pallas import tpu_sc as plsc

E = 320000
D = 128
D2 = D // 2
NC, NS = 2, 16
NW = NC * NS
E_PER_W = E // NW
CH = 80
N_CHUNKS = E_PER_W // CH
N_GROUPS = CH // 16
EU = 8
N_PAIRS = (N_CHUNKS + 1) // 2


def _sc_body(z2_hbm, eidx_hbm, out_hbm,
             eidx_v, out_v, rows0, rows1, sem0, sem1):
    wid = lax.axis_index("s") * NC + lax.axis_index("c")
    base_w = wid * E_PER_W

    lane = lax.broadcasted_iota(jnp.int32, (16,), 0)
    m15 = lane == 15

    pltpu.sync_copy(eidx_hbm.at[pl.ds(2 * base_w, 2 * E_PER_W)], eidx_v)

    def fetch(i, rows, sem):
        sl = pl.ds(2 * i * CH, 2 * CH)
        pltpu.async_copy(z2_hbm.at[eidx_v.at[sl]], rows, sem)

    def drain(i, rows, sem):
        sl = pl.ds(2 * i * CH, 2 * CH)
        pltpu.make_async_copy(z2_hbm.at[eidx_v.at[sl]], rows, sem).wait()

    def compute(i, rows):
        @plsc.parallel_loop(0, CH, step=1, unroll=EU)
        def edge_body(e):
            f32s = []
            for c in range(D2 // 16):
                s = rows[2 * e, pl.ds(c * 16, 16)]
                d = rows[2 * e + 1, pl.ds(c * 16, 16)]
                p = plsc.bitcast(s, jnp.bfloat16) * plsc.bitcast(d, jnp.bfloat16)
                plo, phi = plsc.unpack(p, format=plsc.PackFormat.INTERLEAVED)
                f32s.append(plo)
                f32s.append(phi)
            acc = ((f32s[0] + f32s[1]) + (f32s[2] + f32s[3])) + \
                  ((f32s[4] + f32s[5]) + (f32s[6] + f32s[7]))
            tot = plsc.cumsum(acc)
            plsc.store_scatter(
                out_v, [jnp.zeros((16,), jnp.int32) + (i * CH + e)],
                tot, mask=m15)

        def sig_body(g, _):
            sl = pl.ds(i * CH + g * 16, 16)
            v = out_v[sl]
            out_v[sl] = 1.0 / (1.0 + jnp.exp(-v))
            return 0

        lax.fori_loop(0, N_GROUPS, sig_body, 0)

    fetch(0, rows0, sem0)

    def pair_body(j, _):
        i0 = 2 * j
        i1 = 2 * j + 1

        @pl.when(i1 < N_CHUNKS)
        def _():
            fetch(i1, rows1, sem1)

        drain(i0, rows0, sem0)
        compute(i0, rows0)

        @pl.when(i0 + 2 < N_CHUNKS)
        def _():
            fetch(i0 + 2, rows0, sem0)

        @pl.when(i1 < N_CHUNKS)
        def _():
            drain(i1, rows1, sem1)
            compute(i1, rows1)

        return 0

    lax.fori_loop(0, N_PAIRS, pair_body, 0)
    pltpu.sync_copy(out_v, out_hbm.at[pl.ds(base_w, E_PER_W)])


@jax.jit
def _decode(z, edge_index):
    zb = z.astype(jnp.bfloat16)
    z2 = jax.lax.bitcast_convert_type(
        zb.reshape(z.shape[0], D2, 2), jnp.int32)
    eidx = edge_index.T.reshape(-1)
    mesh = plsc.VectorSubcoreMesh(core_axis_name="c", subcore_axis_name="s")
    fn = pl.kernel(
        _sc_body,
        out_type=jax.ShapeDtypeStruct((E,), jnp.float32),
        mesh=mesh,
        scratch_types=[
            pltpu.VMEM((2 * E_PER_W,), jnp.int32),
            pltpu.VMEM((E_PER_W,), jnp.float32),
            pltpu.VMEM((2 * CH, D2), jnp.int32),
            pltpu.VMEM((2 * CH, D2), jnp.int32),
            pltpu.SemaphoreType.DMA,
            pltpu.SemaphoreType.DMA,
        ],
        compiler_params=pltpu.CompilerParams(
            needs_layout_passes=False,
            use_tc_tiling_on_sc=False,
        ),
    )
    return fn(z2, eidx)


def kernel(z, edge_index):
    return _decode(z, edge_index)


# speedup vs baseline: 2.2534x; 2.2534x over previous
import jax
import jax.numpy as jnp
from jax import lax
from jax.experimental import pallas as pl
from jax.experimental.pallas import tpu as pltpu
from jax.experimental.pallas import tpu_sc as plsc

E = 320000
D = 128
D2 = D // 2
NC, NS = 2, 16
NW = NC * NS
E_PER_W = E // NW
CH = 400
N_CHUNKS = E_PER_W // CH
EU = 8
N_PAIRS = N_CHUNKS // 2
N_SIG = E_PER_W // 16


def _sc_body(z2_hbm, src_hbm, dst_hbm, out_hbm,
             out_v, si0, di0, si1, di1, srows0, drows0, srows1, drows1,
             isem0, isem1, rsem0, rsem1):
    wid = lax.axis_index("s") * NC + lax.axis_index("c")
    base_w = wid * E_PER_W

    lane = lax.broadcasted_iota(jnp.int32, (16,), 0)
    m15 = lane == 15

    def idx_fetch(i, si, di, isem):
        sl = pl.ds(base_w + i * CH, CH)
        pltpu.async_copy(src_hbm.at[sl], si, isem)
        pltpu.async_copy(dst_hbm.at[sl], di, isem)

    def idx_drain(i, si, di, isem):
        sl = pl.ds(base_w + i * CH, CH)
        pltpu.make_async_copy(src_hbm.at[sl], si, isem).wait()
        pltpu.make_async_copy(dst_hbm.at[sl], di, isem).wait()

    def row_fetch(si, di, srows, drows, rsem):
        pltpu.async_copy(z2_hbm.at[si], srows, rsem)
        pltpu.async_copy(z2_hbm.at[di], drows, rsem)

    def row_drain(si, di, srows, drows, rsem):
        pltpu.make_async_copy(z2_hbm.at[si], srows, rsem).wait()
        pltpu.make_async_copy(z2_hbm.at[di], drows, rsem).wait()

    def compute(i, srows, drows):
        @plsc.parallel_loop(0, CH, step=1, unroll=EU)
        def edge_body(e):
            f32s = []
            for c in range(D2 // 16):
                s = srows[e, pl.ds(c * 16, 16)]
                d = drows[e, pl.ds(c * 16, 16)]
                p = plsc.bitcast(s, jnp.bfloat16) * plsc.bitcast(d, jnp.bfloat16)
                plo, phi = plsc.unpack(p, format=plsc.PackFormat.INTERLEAVED)
                f32s.append(plo)
                f32s.append(phi)
            acc = ((f32s[0] + f32s[1]) + (f32s[2] + f32s[3])) + \
                  ((f32s[4] + f32s[5]) + (f32s[6] + f32s[7]))
            tot = plsc.cumsum(acc)
            plsc.store_scatter(
                out_v, [jnp.zeros((16,), jnp.int32) + (i * CH + e)],
                tot, mask=m15)

    idx_fetch(0, si0, di0, isem0)
    idx_fetch(1, si1, di1, isem1)
    idx_drain(0, si0, di0, isem0)
    row_fetch(si0, di0, srows0, drows0, rsem0)

    def pair_body(j, _):
        i0 = 2 * j
        i1 = 2 * j + 1

        @pl.when(i1 < N_CHUNKS)
        def _():
            idx_drain(i1, si1, di1, isem1)
            row_fetch(si1, di1, srows1, drows1, rsem1)

        row_drain(si0, di0, srows0, drows0, rsem0)

        @pl.when(i0 + 2 < N_CHUNKS)
        def _():
            idx_fetch(i0 + 2, si0, di0, isem0)

        compute(i0, srows0, drows0)

        @pl.when(i0 + 2 < N_CHUNKS)
        def _():
            idx_drain(i0 + 2, si0, di0, isem0)
            row_fetch(si0, di0, srows0, drows0, rsem0)

        @pl.when(i1 < N_CHUNKS)
        def _():
            row_drain(si1, di1, srows1, drows1, rsem1)

            @pl.when(i1 + 2 < N_CHUNKS)
            def _():
                idx_fetch(i1 + 2, si1, di1, isem1)

            compute(i1, srows1, drows1)

        return 0

    lax.fori_loop(0, (N_CHUNKS + 1) // 2, pair_body, 0)

    def sig_body(g, _):
        sl = pl.ds(g * 16, 16)
        v = out_v[sl]
        out_v[sl] = 1.0 / (1.0 + jnp.exp(-v))
        return 0

    lax.fori_loop(0, N_SIG, sig_body, 0)
    pltpu.sync_copy(out_v, out_hbm.at[pl.ds(base_w, E_PER_W)])


@jax.jit
def _decode(z, src, dst):
    zb = z.astype(jnp.bfloat16)
    z2 = jax.lax.bitcast_convert_type(
        zb.reshape(z.shape[0], D2, 2), jnp.int32)
    mesh = plsc.VectorSubcoreMesh(core_axis_name="c", subcore_axis_name="s")
    fn = pl.kernel(
        _sc_body,
        out_type=jax.ShapeDtypeStruct((E,), jnp.float32),
        mesh=mesh,
        scratch_types=[
            pltpu.VMEM((E_PER_W,), jnp.float32),
            pltpu.VMEM((CH,), jnp.int32),
            pltpu.VMEM((CH,), jnp.int32),
            pltpu.VMEM((CH,), jnp.int32),
            pltpu.VMEM((CH,), jnp.int32),
            pltpu.VMEM((CH, D2), jnp.int32),
            pltpu.VMEM((CH, D2), jnp.int32),
            pltpu.VMEM((CH, D2), jnp.int32),
            pltpu.VMEM((CH, D2), jnp.int32),
            pltpu.SemaphoreType.DMA,
            pltpu.SemaphoreType.DMA,
            pltpu.SemaphoreType.DMA,
            pltpu.SemaphoreType.DMA,
        ],
        compiler_params=pltpu.CompilerParams(
            needs_layout_passes=False,
            use_tc_tiling_on_sc=False,
        ),
    )
    return fn(z2, src, dst)


def kernel(z, edge_index):
    return _decode(z, edge_index[0], edge_index[1])
